# baseline (speedup 1.0000x reference)
import functools

import numpy as np

import jax
import jax.numpy as jnp
from jax import lax
from jax.experimental import pallas as pl
from jax.experimental.pallas import tpu as pltpu
from jax.experimental.pallas import tpu_sc as plsc

N = 8192
V = 8192
K = 2048
TN = 256
GRID = N // TN
L = 16
NVEC = N // L

_INTERPRET = False


def _stats_body(targets_ref, logits_ref, keys_ref, nll_ref):
    x = logits_ref[...]
    t = targets_ref[...]
    col = lax.broadcasted_iota(jnp.int32, (TN, V), 1)
    tmask = col == t
    m = jnp.max(x, axis=1, keepdims=True)
    xt = jnp.sum(jnp.where(tmask, x, 0.0), axis=1, keepdims=True)
    xm = x - m
    e = jnp.exp(xm)
    s = jnp.sum(e, axis=1, keepdims=True)
    w = jnp.sum(e * xm, axis=1, keepdims=True)
    logs = jnp.log(s)
    ent = logs - w / s
    nll = (m + logs) - xt
    b = lax.bitcast_convert_type(ent, jnp.int32)
    keys_ref[...] = jnp.where(b < 0, b ^ 0x7FFFFFFF, b)
    nll_ref[...] = nll


def _stage1(logits_flat, targets_col):
    return pl.pallas_call(
        _stats_body,
        grid=(GRID,),
        in_specs=[
            pl.BlockSpec((TN, 1), lambda i: (i, 0)),
            pl.BlockSpec((TN, V), lambda i: (i, 0)),
        ],
        out_specs=[
            pl.BlockSpec((TN, 1), lambda i: (i, 0)),
            pl.BlockSpec((TN, 1), lambda i: (i, 0)),
        ],
        out_shape=[
            jax.ShapeDtypeStruct((N, 1), jnp.int32),
            jax.ShapeDtypeStruct((N, 1), jnp.float32),
        ],
        interpret=_INTERPRET,
    )(targets_col, logits_flat)


_SIGN = int(np.int32(np.uint32(0x80000000)))


def _select_body(keys_hbm, nll_hbm, out_hbm, keys_v, nll_v, hist_v, out_v):
    @pl.when((lax.axis_index("c") == 0) & (lax.axis_index("s") == 0))
    def _():
        pltpu.sync_copy(keys_hbm, keys_v)
        pltpu.sync_copy(nll_hbm, nll_v)
        lane = lax.iota(jnp.int32, L)
        ones16 = jnp.full((L,), 1, jnp.int32)
        shiftv = lambda n: jnp.full((L,), n, jnp.int32)

        prefix = jnp.int32(0)
        r = jnp.int32(K)
        for p in range(4):
            shift = 24 - 8 * p
            hmask = int(np.int32(np.uint32((0xFFFFFFFF << (shift + 8))
                                           & 0xFFFFFFFF))) if p else 0

            def zero_body(j, carry):
                hist_v[pl.ds(j * L, L)] = jnp.zeros((L,), jnp.int32)
                return carry

            lax.fori_loop(0, 256, zero_body, 0)

            pref_c = prefix

            def hist_body(i, carry):
                kv = keys_v[pl.ds(i * L, L)]
                ub = kv ^ _SIGN
                mch = (ub & hmask) == pref_c
                digit = lax.shift_right_logical(ub, shiftv(shift)) & 255
                plsc.addupdate_scatter(hist_v, [digit * L + lane], ones16,
                                       mask=mch)
                return carry

            lax.fori_loop(0, NVEC, hist_body, 0)

            def scan_body(j, carry):
                c, dstar, found = carry
                d = 255 - j
                h = jnp.sum(hist_v[pl.ds(d * L, L)])
                done = jnp.logical_and(found == 0, c + h >= r)
                dstar = jnp.where(done, d, dstar)
                c = jnp.where(jnp.logical_or(found == 1, done), c, c + h)
                found = jnp.where(done, jnp.int32(1), found)
                return (c, dstar, found)

            c, dstar, _ = lax.fori_loop(
                0, 256, scan_body,
                (jnp.int32(0), jnp.int32(0), jnp.int32(0)))
            r = r - c
            prefix = prefix | lax.shift_left(dstar, shift)

        t_s = prefix ^ _SIGN

        def fin_body(i, carry):
            acc, tcnt = carry
            kv = keys_v[pl.ds(i * L, L)]
            nv = nll_v[pl.ds(i * L, L)]
            gt = kv > t_s
            eq = kv == t_s
            acc = acc + jnp.sum(jnp.where(gt, nv, 0.0))
            eqc = eq.astype(jnp.int32)
            incl = jnp.cumsum(eqc) + tcnt
            take = jnp.logical_and(eq, incl <= r)
            acc = acc + jnp.sum(jnp.where(take, nv, 0.0))
            tcnt = tcnt + jnp.sum(eqc)
            return (acc, tcnt)

        acc, _ = lax.fori_loop(0, NVEC, fin_body,
                               (jnp.float32(0), jnp.int32(0)))
        out_v[...] = jnp.full((L,), acc * (1.0 / K), jnp.float32)
        pltpu.sync_copy(out_v, out_hbm)


_select = pl.kernel(
    _select_body,
    out_type=jax.ShapeDtypeStruct((L,), jnp.float32),
    mesh=plsc.VectorSubcoreMesh(core_axis_name="c", subcore_axis_name="s"),
    compiler_params=pltpu.CompilerParams(needs_layout_passes=False),
    scratch_types=[
        pltpu.VMEM((N,), jnp.int32),
        pltpu.VMEM((N,), jnp.float32),
        pltpu.VMEM((256 * L,), jnp.int32),
        pltpu.VMEM((L,), jnp.float32),
    ],
)


def kernel(logits, targets):
    logits_flat = logits.reshape(N, V)
    targets_col = targets.reshape(N, 1)
    keys, nll = _stage1(logits_flat, targets_col)
    sel = _select(keys.reshape(N), nll.reshape(N))
    return sel[0]
